# baseline (speedup 1.0000x reference)
import functools

import numpy as np
import jax
import jax.numpy as jnp
from jax.experimental import pallas as pl
from jax.experimental.pallas import tpu as pltpu

_BF = jnp.bfloat16
_F32 = jnp.float32
_CONTRACT0 = (((0,), (0,)), ((), ()))


def _resize_matrix(in_size, out_size):
    scale = in_size / out_size
    dst = np.arange(out_size)
    src = np.clip((dst + 0.5) * scale - 0.5, 0.0, None)
    i0 = np.minimum(np.floor(src).astype(np.int64), in_size - 1)
    i1 = np.minimum(i0 + 1, in_size - 1)
    w1 = src - i0
    m = np.zeros((out_size, in_size), np.float32)
    m[np.arange(out_size), i0] += 1.0 - w1
    m[np.arange(out_size), i1] += w1
    return m


def _upsample_kron(in_hw, out_hw):
    return np.kron(_resize_matrix(in_hw[0], out_hw[0]),
                   _resize_matrix(in_hw[1], out_hw[1]))


def _stride2_select(H, W):
    Wp = W + 2
    Ho, Wo = (H + 1) // 2, (W + 1) // 2
    s = np.zeros((Ho * Wo, H * Wp), np.float32)
    for ho in range(Ho):
        for wo in range(Wo):
            s[ho * Wo + wo, (2 * ho) * Wp + 2 * wo] = 1.0
    return s


def _conv9(pad_ref, H, W, w_ref, b_ref, relu):
    Wp = W + 2
    A = H * Wp
    acc = jnp.dot(pad_ref[pl.ds(0, A), :], w_ref[0],
                  preferred_element_type=_F32)
    for t in range(1, 9):
        off = (t // 3) * Wp + (t % 3)
        acc = acc + jnp.dot(pad_ref[pl.ds(off, A), :], w_ref[t],
                            preferred_element_type=_F32)
    acc = acc + b_ref[...]
    if relu:
        acc = jnp.maximum(acc, 0.0)
    return acc


def _fill_pad(pad_ref, v, H, W):
    Wp = W + 2
    vb = v.astype(pad_ref.dtype)
    for i in range(H):
        pad_ref[pl.ds((i + 1) * Wp + 1, W), :] = vb[i * W:(i + 1) * W, :]


def _fill_pad_from_anchors(pad_ref, acc, H, W):
    Wp = W + 2
    for i in range(H):
        pad_ref[pl.ds((i + 1) * Wp + 1, W), :] = (
            acc[i * Wp:i * Wp + W, :].astype(pad_ref.dtype))


def _crop_store(acc, out_ref, H, W):
    Wp = W + 2
    for i in range(H):
        out_ref[0, pl.ds(i * W, W), :] = acc[i * Wp:i * Wp + W, :]


def _fpn_kernel(x0_ref, x1_ref, x2_ref,
                lw0_ref, lw1_ref, lw2_ref,
                lb0_ref, lb1_ref, lb2_ref,
                pw0_ref, pw1_ref, pw2_ref,
                pb0_ref, pb1_ref, pb2_ref,
                dw1_ref, db1_ref, dw2_ref, db2_ref,
                m1_ref, m2_ref, s1_ref, s2_ref,
                p1_ref, p2_ref, p3_ref, d1_ref, d2_ref,
                padA_ref, padB_ref, padC_ref, padD_ref,
                lw0_s, lw1_s, lw2_s, pw0_s, pw1_s, pw2_s, dw1_s, dw2_s,
                *, H0, W0, H1, W1, H2, W2):
    Hd1, Wd1 = (H2 + 1) // 2, (W2 + 1) // 2
    first_step = pl.program_id(1) == 0

    @pl.when(first_step)
    def _stage():
        lw0_s[...] = lw0_ref[...].astype(_BF)
        lw1_s[...] = lw1_ref[...].astype(_BF)
        lw2_s[...] = lw2_ref[...].astype(_BF)
        pw0_s[...] = pw0_ref[...].astype(_BF)
        pw1_s[...] = pw1_ref[...].astype(_BF)
        pw2_s[...] = pw2_ref[...].astype(_BF)
        dw1_s[...] = dw1_ref[...].astype(_BF)
        dw2_s[...] = dw2_ref[...].astype(_BF)
        for r in (padA_ref, padB_ref, padC_ref, padD_ref):
            r[...] = jnp.zeros(r.shape, r.dtype)

    out3 = jnp.dot(x2_ref[0], lw0_s[...],
                   preferred_element_type=_F32) + lb0_ref[...]
    up2 = jnp.dot(m2_ref[...], out3.astype(_BF),
                  preferred_element_type=_F32)
    out2 = jnp.dot(x1_ref[0], lw1_s[...],
                   preferred_element_type=_F32) + up2 + lb1_ref[...]
    up1 = jnp.dot(m1_ref[...], out2.astype(_BF),
                  preferred_element_type=_F32)
    out1 = jnp.dot(x0_ref[0], lw2_s[...],
                   preferred_element_type=_F32) + up1 + lb2_ref[...]

    _fill_pad(padC_ref, out3, H2, W2)
    acc3 = _conv9(padC_ref, H2, W2, pw2_s, pb2_ref, relu=True)
    _crop_store(acc3, p3_ref, H2, W2)

    _fill_pad_from_anchors(padC_ref, acc3, H2, W2)
    accd1 = _conv9(padC_ref, H2, W2, dw1_s, db1_ref, relu=False)
    d1v = jnp.dot(s1_ref[...], accd1, preferred_element_type=_F32)
    d1_ref[0] = d1v

    _fill_pad(padD_ref, d1v, Hd1, Wd1)
    accd2 = _conv9(padD_ref, Hd1, Wd1, dw2_s, db2_ref, relu=False)
    d2_ref[0] = jnp.dot(s2_ref[...], accd2, preferred_element_type=_F32)

    _fill_pad(padB_ref, out2, H1, W1)
    acc2 = _conv9(padB_ref, H1, W1, pw1_s, pb1_ref, relu=True)
    _crop_store(acc2, p2_ref, H1, W1)

    _fill_pad(padA_ref, out1, H0, W0)
    acc1 = _conv9(padA_ref, H0, W0, pw0_s, pb0_ref, relu=True)
    _crop_store(acc1, p1_ref, H0, W0)


def kernel(r0, r1, r2,
           lat_w_0, lat_w_1, lat_w_2,
           lat_b_0, lat_b_1, lat_b_2,
           pred_w_0, pred_w_1, pred_w_2,
           pred_b_0, pred_b_1, pred_b_2,
           down1_w, down1_b, down2_w, down2_b):
    N, C0, H0, W0 = r0.shape
    _, C1, H1, W1 = r1.shape
    _, C2, H2, W2 = r2.shape
    Co = lat_w_0.shape[1]
    Hd1, Wd1 = (H2 + 1) // 2, (W2 + 1) // 2
    Hd2, Wd2 = (Hd1 + 1) // 2, (Wd1 + 1) // 2

    x0 = jnp.transpose(r0, (0, 2, 3, 1)).reshape(N, H0 * W0, C0).astype(_BF)
    x1 = jnp.transpose(r1, (0, 2, 3, 1)).reshape(N, H1 * W1, C1).astype(_BF)
    x2 = jnp.transpose(r2, (0, 2, 3, 1)).reshape(N, H2 * W2, C2).astype(_BF)

    m1 = jnp.asarray(_upsample_kron((H1, W1), (H0, W0)), dtype=_BF)
    m2 = jnp.asarray(_upsample_kron((H2, W2), (H1, W1)), dtype=_BF)
    s1 = jnp.asarray(_stride2_select(H2, W2))
    s2 = jnp.asarray(_stride2_select(Hd1, Wd1))

    assert N % 2 == 0
    NB = N // 2

    def batch3(c, m):
        return pl.BlockSpec((1, c, m), lambda i, j: (i * NB + j, 0, 0))

    def const2(a, b):
        return pl.BlockSpec((a, b), lambda i, j: (0, 0))

    def const3(a, b, c):
        return pl.BlockSpec((a, b, c), lambda i, j: (0, 0, 0))

    flops_lat = 2 * N * Co * (C0 * H0 * W0 + C1 * H1 * W1 + C2 * H2 * W2)
    flops_up = 2 * N * Co * (H0 * W0 * H1 * W1 + H1 * W1 * H2 * W2)
    flops_conv = 2 * N * 9 * Co * Co * (
        H0 * (W0 + 2) + H1 * (W1 + 2) + 2 * H2 * (W2 + 2) + Hd1 * (Wd1 + 2))

    out_shapes = [
        jax.ShapeDtypeStruct((N, H0 * W0, Co), _F32),
        jax.ShapeDtypeStruct((N, H1 * W1, Co), _F32),
        jax.ShapeDtypeStruct((N, H2 * W2, Co), _F32),
        jax.ShapeDtypeStruct((N, Hd1 * Wd1, Co), _F32),
        jax.ShapeDtypeStruct((N, Hd2 * Wd2, Co), _F32),
    ]

    p1, p2, p3, d1, d2 = pl.pallas_call(
        functools.partial(_fpn_kernel, H0=H0, W0=W0, H1=H1, W1=W1,
                          H2=H2, W2=W2),
        out_shape=out_shapes,
        grid=(2, NB),
        in_specs=[
            batch3(H0 * W0, C0), batch3(H1 * W1, C1), batch3(H2 * W2, C2),
            const2(C2, Co), const2(C1, Co), const2(C0, Co),
            const2(1, Co), const2(1, Co), const2(1, Co),
            const3(9, Co, Co), const3(9, Co, Co), const3(9, Co, Co),
            const2(1, Co), const2(1, Co), const2(1, Co),
            const3(9, Co, Co), const2(1, Co),
            const3(9, Co, Co), const2(1, Co),
            const2(H0 * W0, H1 * W1), const2(H1 * W1, H2 * W2),
            const2(Hd1 * Wd1, H2 * (W2 + 2)),
            const2(Hd2 * Wd2, Hd1 * (Wd1 + 2)),
        ],
        out_specs=[
            batch3(H0 * W0, Co), batch3(H1 * W1, Co), batch3(H2 * W2, Co),
            batch3(Hd1 * Wd1, Co), batch3(Hd2 * Wd2, Co),
        ],
        scratch_shapes=[
            pltpu.VMEM(((H0 + 2) * (W0 + 2) + 2, Co), _BF),
            pltpu.VMEM(((H1 + 2) * (W1 + 2) + 2, Co), _BF),
            pltpu.VMEM(((H2 + 2) * (W2 + 2) + 2, Co), _BF),
            pltpu.VMEM(((Hd1 + 2) * (Wd1 + 2) + 2, Co), _BF),
            pltpu.VMEM((C2, Co), _BF),
            pltpu.VMEM((C1, Co), _BF),
            pltpu.VMEM((C0, Co), _BF),
            pltpu.VMEM((9, Co, Co), _BF),
            pltpu.VMEM((9, Co, Co), _BF),
            pltpu.VMEM((9, Co, Co), _BF),
            pltpu.VMEM((9, Co, Co), _BF),
            pltpu.VMEM((9, Co, Co), _BF),
        ],
        compiler_params=pltpu.CompilerParams(
            dimension_semantics=("parallel", "arbitrary")),
        cost_estimate=pl.CostEstimate(
            flops=flops_lat + flops_up + flops_conv,
            transcendentals=0,
            bytes_accessed=4 * (N * (C0 * H0 * W0 + C1 * H1 * W1
                                     + C2 * H2 * W2)
                                + N * Co * (H0 * W0 + H1 * W1 + H2 * W2
                                            + Hd1 * Wd1 + Hd2 * Wd2)
                                + Co * (C0 + C1 + C2 + 5 * 9 * Co))),
    )(x0, x1, x2,
      lat_w_0, lat_w_1, lat_w_2,
      lat_b_0, lat_b_1, lat_b_2,
      pred_w_0, pred_w_1, pred_w_2,
      pred_b_0, pred_b_1, pred_b_2,
      down1_w, down1_b, down2_w, down2_b,
      m1, m2, s1, s2)

    def to_nchw(t, h, w):
        return jnp.transpose(t.reshape(N, h, w, Co), (0, 3, 1, 2))

    return [to_nchw(p1, H0, W0), to_nchw(p2, H1, W1), to_nchw(p3, H2, W2),
            to_nchw(d1, Hd1, Wd1), to_nchw(d2, Hd2, Wd2)]


# speedup vs baseline: 1.3259x; 1.3259x over previous
import functools

import numpy as np
import jax
import jax.numpy as jnp
from jax.experimental import pallas as pl
from jax.experimental.pallas import tpu as pltpu

_BF = jnp.bfloat16
_F32 = jnp.float32
_CONTRACT0 = (((0,), (0,)), ((), ()))


def _resize_matrix(in_size, out_size):
    scale = in_size / out_size
    dst = np.arange(out_size)
    src = np.clip((dst + 0.5) * scale - 0.5, 0.0, None)
    i0 = np.minimum(np.floor(src).astype(np.int64), in_size - 1)
    i1 = np.minimum(i0 + 1, in_size - 1)
    w1 = src - i0
    m = np.zeros((out_size, in_size), np.float32)
    m[np.arange(out_size), i0] += 1.0 - w1
    m[np.arange(out_size), i1] += w1
    return m


def _upsample_kron(in_hw, out_hw):
    return np.kron(_resize_matrix(in_hw[0], out_hw[0]),
                   _resize_matrix(in_hw[1], out_hw[1]))


def _stride2_select(H, W):
    Wp = W + 2
    Ho, Wo = (H + 1) // 2, (W + 1) // 2
    s = np.zeros((Ho * Wo, H * Wp), np.float32)
    for ho in range(Ho):
        for wo in range(Wo):
            s[ho * Wo + wo, (2 * ho) * Wp + 2 * wo] = 1.0
    return s


def _conv9(pad_ref, H, W, w_ref, b_ref, relu):
    Wp = W + 2
    A = H * Wp
    acc = jnp.dot(pad_ref[pl.ds(0, A), :], w_ref[0],
                  preferred_element_type=_F32)
    for t in range(1, 9):
        off = (t // 3) * Wp + (t % 3)
        acc = acc + jnp.dot(pad_ref[pl.ds(off, A), :], w_ref[t],
                            preferred_element_type=_F32)
    acc = acc + b_ref[...]
    if relu:
        acc = jnp.maximum(acc, 0.0)
    return acc


def _fill_pad(pad_ref, v, H, W):
    Wp = W + 2
    vb = v.astype(pad_ref.dtype)
    for i in range(H):
        pad_ref[pl.ds((i + 1) * Wp + 1, W), :] = vb[i * W:(i + 1) * W, :]


def _fill_pad_from_anchors(pad_ref, acc, H, W):
    Wp = W + 2
    for i in range(H):
        pad_ref[pl.ds((i + 1) * Wp + 1, W), :] = (
            acc[i * Wp:i * Wp + W, :].astype(pad_ref.dtype))


def _crop_store(acc, out_ref, H, W):
    Wp = W + 2
    for i in range(H):
        out_ref[0, pl.ds(i * W, W), :] = acc[i * Wp:i * Wp + W, :]


def _fpn_kernel(x0_ref, x1_ref, x2_ref,
                lw0_ref, lw1_ref, lw2_ref,
                lb0_ref, lb1_ref, lb2_ref,
                pw0_ref, pw1_ref, pw2_ref,
                pb0_ref, pb1_ref, pb2_ref,
                dw1_ref, db1_ref, dw2_ref, db2_ref,
                m1_ref, m2_ref, s1_ref, s2_ref,
                p1_ref, p2_ref, p3_ref, d1_ref, d2_ref,
                padA_ref, padB_ref, padC_ref, padD_ref,
                lw0_s, lw1_s, lw2_s, pw0_s, pw1_s, pw2_s, dw1_s, dw2_s,
                *, H0, W0, H1, W1, H2, W2):
    Hd1, Wd1 = (H2 + 1) // 2, (W2 + 1) // 2
    first_step = pl.program_id(1) == 0

    @pl.when(first_step)
    def _stage():
        lw0_s[...] = lw0_ref[...].astype(_BF)
        lw1_s[...] = lw1_ref[...].astype(_BF)
        lw2_s[...] = lw2_ref[...].astype(_BF)
        pw0_s[...] = pw0_ref[...].astype(_BF)
        pw1_s[...] = pw1_ref[...].astype(_BF)
        pw2_s[...] = pw2_ref[...].astype(_BF)
        dw1_s[...] = dw1_ref[...].astype(_BF)
        dw2_s[...] = dw2_ref[...].astype(_BF)
        for r in (padA_ref, padB_ref, padC_ref, padD_ref):
            r[...] = jnp.zeros(r.shape, r.dtype)

    out3 = jnp.dot(x2_ref[0].astype(_BF), lw0_s[...],
                   preferred_element_type=_F32) + lb0_ref[...]
    up2 = jnp.dot(m2_ref[...], out3.astype(_BF),
                  preferred_element_type=_F32)
    out2 = jnp.dot(x1_ref[0].astype(_BF), lw1_s[...],
                   preferred_element_type=_F32) + up2 + lb1_ref[...]
    up1 = jnp.dot(m1_ref[...], out2.astype(_BF),
                  preferred_element_type=_F32)
    out1 = jnp.dot(x0_ref[0].astype(_BF), lw2_s[...],
                   preferred_element_type=_F32) + up1 + lb2_ref[...]

    _fill_pad(padC_ref, out3, H2, W2)
    acc3 = _conv9(padC_ref, H2, W2, pw2_s, pb2_ref, relu=True)
    _crop_store(acc3, p3_ref, H2, W2)

    _fill_pad_from_anchors(padC_ref, acc3, H2, W2)
    accd1 = _conv9(padC_ref, H2, W2, dw1_s, db1_ref, relu=False)
    d1v = jnp.dot(s1_ref[...], accd1, preferred_element_type=_F32)
    d1_ref[0] = d1v

    _fill_pad(padD_ref, d1v, Hd1, Wd1)
    accd2 = _conv9(padD_ref, Hd1, Wd1, dw2_s, db2_ref, relu=False)
    d2_ref[0] = jnp.dot(s2_ref[...], accd2, preferred_element_type=_F32)

    _fill_pad(padB_ref, out2, H1, W1)
    acc2 = _conv9(padB_ref, H1, W1, pw1_s, pb1_ref, relu=True)
    _crop_store(acc2, p2_ref, H1, W1)

    _fill_pad(padA_ref, out1, H0, W0)
    acc1 = _conv9(padA_ref, H0, W0, pw0_s, pb0_ref, relu=True)
    _crop_store(acc1, p1_ref, H0, W0)


def kernel(r0, r1, r2,
           lat_w_0, lat_w_1, lat_w_2,
           lat_b_0, lat_b_1, lat_b_2,
           pred_w_0, pred_w_1, pred_w_2,
           pred_b_0, pred_b_1, pred_b_2,
           down1_w, down1_b, down2_w, down2_b):
    N, C0, H0, W0 = r0.shape
    _, C1, H1, W1 = r1.shape
    _, C2, H2, W2 = r2.shape
    Co = lat_w_0.shape[1]
    Hd1, Wd1 = (H2 + 1) // 2, (W2 + 1) // 2
    Hd2, Wd2 = (Hd1 + 1) // 2, (Wd1 + 1) // 2

    x0 = jnp.transpose(r0, (0, 2, 3, 1)).reshape(N, H0 * W0, C0)
    x1 = jnp.transpose(r1, (0, 2, 3, 1)).reshape(N, H1 * W1, C1)
    x2 = jnp.transpose(r2, (0, 2, 3, 1)).reshape(N, H2 * W2, C2)

    m1 = jnp.asarray(_upsample_kron((H1, W1), (H0, W0)), dtype=_BF)
    m2 = jnp.asarray(_upsample_kron((H2, W2), (H1, W1)), dtype=_BF)
    s1 = jnp.asarray(_stride2_select(H2, W2))
    s2 = jnp.asarray(_stride2_select(Hd1, Wd1))

    assert N % 2 == 0
    NB = N // 2

    def batch3(c, m):
        return pl.BlockSpec((1, c, m), lambda i, j: (i * NB + j, 0, 0))

    def const2(a, b):
        return pl.BlockSpec((a, b), lambda i, j: (0, 0))

    def const3(a, b, c):
        return pl.BlockSpec((a, b, c), lambda i, j: (0, 0, 0))

    flops_lat = 2 * N * Co * (C0 * H0 * W0 + C1 * H1 * W1 + C2 * H2 * W2)
    flops_up = 2 * N * Co * (H0 * W0 * H1 * W1 + H1 * W1 * H2 * W2)
    flops_conv = 2 * N * 9 * Co * Co * (
        H0 * (W0 + 2) + H1 * (W1 + 2) + 2 * H2 * (W2 + 2) + Hd1 * (Wd1 + 2))

    out_shapes = [
        jax.ShapeDtypeStruct((N, H0 * W0, Co), _F32),
        jax.ShapeDtypeStruct((N, H1 * W1, Co), _F32),
        jax.ShapeDtypeStruct((N, H2 * W2, Co), _F32),
        jax.ShapeDtypeStruct((N, Hd1 * Wd1, Co), _F32),
        jax.ShapeDtypeStruct((N, Hd2 * Wd2, Co), _F32),
    ]

    p1, p2, p3, d1, d2 = pl.pallas_call(
        functools.partial(_fpn_kernel, H0=H0, W0=W0, H1=H1, W1=W1,
                          H2=H2, W2=W2),
        out_shape=out_shapes,
        grid=(2, NB),
        in_specs=[
            batch3(H0 * W0, C0), batch3(H1 * W1, C1), batch3(H2 * W2, C2),
            const2(C2, Co), const2(C1, Co), const2(C0, Co),
            const2(1, Co), const2(1, Co), const2(1, Co),
            const3(9, Co, Co), const3(9, Co, Co), const3(9, Co, Co),
            const2(1, Co), const2(1, Co), const2(1, Co),
            const3(9, Co, Co), const2(1, Co),
            const3(9, Co, Co), const2(1, Co),
            const2(H0 * W0, H1 * W1), const2(H1 * W1, H2 * W2),
            const2(Hd1 * Wd1, H2 * (W2 + 2)),
            const2(Hd2 * Wd2, Hd1 * (Wd1 + 2)),
        ],
        out_specs=[
            batch3(H0 * W0, Co), batch3(H1 * W1, Co), batch3(H2 * W2, Co),
            batch3(Hd1 * Wd1, Co), batch3(Hd2 * Wd2, Co),
        ],
        scratch_shapes=[
            pltpu.VMEM(((H0 + 2) * (W0 + 2) + 2, Co), _BF),
            pltpu.VMEM(((H1 + 2) * (W1 + 2) + 2, Co), _BF),
            pltpu.VMEM(((H2 + 2) * (W2 + 2) + 2, Co), _BF),
            pltpu.VMEM(((Hd1 + 2) * (Wd1 + 2) + 2, Co), _BF),
            pltpu.VMEM((C2, Co), _BF),
            pltpu.VMEM((C1, Co), _BF),
            pltpu.VMEM((C0, Co), _BF),
            pltpu.VMEM((9, Co, Co), _BF),
            pltpu.VMEM((9, Co, Co), _BF),
            pltpu.VMEM((9, Co, Co), _BF),
            pltpu.VMEM((9, Co, Co), _BF),
            pltpu.VMEM((9, Co, Co), _BF),
        ],
        compiler_params=pltpu.CompilerParams(
            dimension_semantics=("parallel", "arbitrary")),
        cost_estimate=pl.CostEstimate(
            flops=flops_lat + flops_up + flops_conv,
            transcendentals=0,
            bytes_accessed=4 * (N * (C0 * H0 * W0 + C1 * H1 * W1
                                     + C2 * H2 * W2)
                                + N * Co * (H0 * W0 + H1 * W1 + H2 * W2
                                            + Hd1 * Wd1 + Hd2 * Wd2)
                                + Co * (C0 + C1 + C2 + 5 * 9 * Co))),
    )(x0, x1, x2,
      lat_w_0, lat_w_1, lat_w_2,
      lat_b_0, lat_b_1, lat_b_2,
      pred_w_0, pred_w_1, pred_w_2,
      pred_b_0, pred_b_1, pred_b_2,
      down1_w, down1_b, down2_w, down2_b,
      m1, m2, s1, s2)

    def to_nchw(t, h, w):
        return jnp.transpose(t.reshape(N, h, w, Co), (0, 3, 1, 2))

    return [to_nchw(p1, H0, W0), to_nchw(p2, H1, W1), to_nchw(p3, H2, W2),
            to_nchw(d1, Hd1, Wd1), to_nchw(d2, Hd2, Wd2)]
